# baseline (speedup 1.0000x reference)
import functools

import jax
import jax.numpy as jnp
import numpy as np
from jax import lax
from jax.experimental import pallas as pl
from jax.experimental.pallas import tpu as pltpu
from jax.experimental.pallas import tpu_sc as plsc

EMBEDDING_DIM = 256
HALF = EMBEDDING_DIM // 2
NUM_RELATIONS = 64
N_NODES = 10000
N_EDGES = 160000
EPS = 1e-08

NUM_CORES = 2
NUM_SUBCORES = 16
NUM_WORKERS = NUM_CORES * NUM_SUBCORES
LANES = 16

PAD_EDGES = 163840
EDGES_PER_WORKER = PAD_EDGES // NUM_WORKERS
CHUNK = 32
NUM_CHUNKS = EDGES_PER_WORKER // CHUNK
GROUPS = CHUNK // LANES
DIM_GROUPS = HALF // LANES

_RSQRT_MAGIC = np.int32(0x5F3759DF)
_HI_MASK = np.int32(-65536)


def _tc_prep(x_ref, w_ref, xnp_ref, trig_ref):
    x = x_ref[...]
    norm = jnp.sqrt(jnp.sum(x * x, axis=1, keepdims=True))
    xn = x / jnp.maximum(norm, 1e-12)
    re = xn[:, :HALF]
    im = xn[:, HALF:]
    rb = lax.bitcast_convert_type(re, jnp.uint32)
    ib = lax.bitcast_convert_type(im, jnp.uint32)
    rb = rb + 0x7FFF + ((rb >> 16) & 1)
    ib = ib + 0x7FFF + ((ib >> 16) & 1)
    packed = (ib & np.uint32(0xFFFF0000)) | (rb >> 16)
    xnp_ref[...] = lax.bitcast_convert_type(packed, jnp.int32)
    w = w_ref[...]
    cb = lax.bitcast_convert_type(jnp.cos(w), jnp.uint32)
    sb = lax.bitcast_convert_type(jnp.sin(w), jnp.uint32)
    cb = cb + 0x7FFF + ((cb >> 16) & 1)
    sb = sb + 0x7FFF + ((sb >> 16) & 1)
    tpacked = (sb & np.uint32(0xFFFF0000)) | (cb >> 16)
    trig_ref[...] = lax.bitcast_convert_type(tpacked, jnp.int32)


def _approx_sqrt(sq):
    i = lax.bitcast_convert_type(sq, jnp.int32)
    i = _RSQRT_MAGIC - lax.shift_right_logical(i, 1)
    r = lax.bitcast_convert_type(i, jnp.float32)
    r = r * (1.5 - (sq * 0.5) * r * r)
    return sq * r


def _unpack(word):
    lo = lax.bitcast_convert_type(lax.shift_left(word, 16), jnp.float32)
    hi = lax.bitcast_convert_type(word & _HI_MASK, jnp.float32)
    return lo, hi


NBUF = 2
ROWS_PER_SUB = 624


def _sc_body(xnp_hbm, trig_hbm, heads_hbm, tails_hbm, rels_hbm, out_hbm,
             hidx, tidx, ridx, trig_v, xnp_sh,
             hrows0, trows0, hrows1, trows1, oacc,
             sem_h0, sem_t0, sem_h1, sem_t1):
    hbufs = (hrows0, hrows1)
    tbufs = (trows0, trows1)
    hsems = (sem_h0, sem_h1)
    tsems = (sem_t0, sem_t1)
    sid = lax.axis_index("s")
    wid = sid * NUM_CORES + lax.axis_index("c")
    base = wid * EDGES_PER_WORKER
    pltpu.sync_copy(xnp_hbm.at[pl.ds(sid * ROWS_PER_SUB, ROWS_PER_SUB)],
                    xnp_sh.at[pl.ds(sid * ROWS_PER_SUB, ROWS_PER_SUB)])

    @pl.when(sid == 0)
    def _():
        tail = NUM_SUBCORES * ROWS_PER_SUB
        pltpu.sync_copy(xnp_hbm.at[pl.ds(tail, N_NODES - tail)],
                        xnp_sh.at[pl.ds(tail, N_NODES - tail)])

    pltpu.sync_copy(heads_hbm.at[pl.ds(base, EDGES_PER_WORKER)], hidx)
    pltpu.sync_copy(tails_hbm.at[pl.ds(base, EDGES_PER_WORKER)], tidx)
    pltpu.sync_copy(rels_hbm.at[pl.ds(base, EDGES_PER_WORKER)], ridx)

    pltpu.sync_copy(trig_hbm, trig_v)
    plsc.subcore_barrier()
    lane = lax.broadcasted_iota(jnp.int32, (LANES,), 0)

    def issue(ci, hrows, trows, sem_h, sem_t):
        off = ci * CHUNK
        pltpu.async_copy(xnp_sh.at[hidx.at[pl.ds(off, CHUNK)]], hrows, sem_h)
        pltpu.async_copy(xnp_sh.at[tidx.at[pl.ds(off, CHUNK)]], trows, sem_t)

    def wait(hrows, trows, sem_h, sem_t):
        pltpu.make_async_copy(xnp_sh.at[hidx.at[pl.ds(0, CHUNK)]], hrows,
                              sem_h).wait()
        pltpu.make_async_copy(xnp_sh.at[tidx.at[pl.ds(0, CHUNK)]], trows,
                              sem_t).wait()

    def compute(ci, hrows, trows):
        off = ci * CHUNK

        def group_body(gi, gcarry):
            goff = gi * LANES
            relvec = ridx[pl.ds(off + goff, LANES)]
            vec = jnp.zeros((LANES,), jnp.float32)
            for e in range(LANES):
                eidx = goff + e
                rel = relvec[e]
                acc = jnp.zeros((LANES,), jnp.float32)
                for d in range(DIM_GROUPS):
                    sl = pl.ds(d * LANES, LANES)
                    sl_im = pl.ds(HALF + d * LANES, LANES)
                    hre, him = _unpack(hrows[eidx, sl])
                    tre, tim = _unpack(trows[eidx, sl])
                    cw, sw = _unpack(trig_v[rel, sl])
                    re_d = hre * cw - him * sw - tre
                    im_d = hre * sw + him * cw - tim
                    sq = re_d * re_d + im_d * im_d + EPS
                    acc = acc + _approx_sqrt(sq)
                s = jnp.sum(acc * (-1.0 / HALF))
                vec = jnp.where(lane == e, s, vec)
            oacc[pl.ds(off + goff, LANES)] = vec
            return gcarry

        lax.fori_loop(0, GROUPS, group_body, 0)

    for b in range(NBUF):
        issue(b, hbufs[b], tbufs[b], hsems[b], tsems[b])

    def quad_body(qi, carry):
        ci = qi * NBUF
        for b in range(NBUF):
            wait(hbufs[b], tbufs[b], hsems[b], tsems[b])
            compute(ci + b, hbufs[b], tbufs[b])

            @pl.when(qi < NUM_CHUNKS // NBUF - 1)
            def _():
                issue(ci + b + NBUF, hbufs[b], tbufs[b], hsems[b], tsems[b])

        return carry

    lax.fori_loop(0, NUM_CHUNKS // NBUF, quad_body, 0)
    pltpu.sync_copy(oacc, out_hbm.at[pl.ds(base, EDGES_PER_WORKER)])


_sc_score = functools.partial(
    pl.kernel,
    out_type=jax.ShapeDtypeStruct((PAD_EDGES,), jnp.float32),
    mesh=plsc.VectorSubcoreMesh(core_axis_name="c", subcore_axis_name="s"),
    compiler_params=pltpu.CompilerParams(needs_layout_passes=False),
    scratch_types=[
        pltpu.VMEM((EDGES_PER_WORKER,), jnp.int32),
        pltpu.VMEM((EDGES_PER_WORKER,), jnp.int32),
        pltpu.VMEM((EDGES_PER_WORKER,), jnp.int32),
        pltpu.VMEM((NUM_RELATIONS, HALF), jnp.int32),
        pltpu.VMEM_SHARED((N_NODES, HALF), jnp.int32),
        pltpu.VMEM((CHUNK, HALF), jnp.int32),
        pltpu.VMEM((CHUNK, HALF), jnp.int32),
        pltpu.VMEM((CHUNK, HALF), jnp.int32),
        pltpu.VMEM((CHUNK, HALF), jnp.int32),
        pltpu.VMEM((EDGES_PER_WORKER,), jnp.float32),
        pltpu.SemaphoreType.DMA,
        pltpu.SemaphoreType.DMA,
        pltpu.SemaphoreType.DMA,
        pltpu.SemaphoreType.DMA,
    ],
)(_sc_body)


def kernel(node_embeddings, edge_index, relation_type, rel_weight):
    xnp, trig = pl.pallas_call(
        _tc_prep,
        out_shape=[
            jax.ShapeDtypeStruct((N_NODES, HALF), jnp.int32),
            jax.ShapeDtypeStruct((NUM_RELATIONS, HALF), jnp.int32),
        ],
    )(node_embeddings, rel_weight)
    pad = PAD_EDGES - N_EDGES
    heads = jnp.pad(edge_index[0].astype(jnp.int32), (0, pad))
    tails = jnp.pad(edge_index[1].astype(jnp.int32), (0, pad))
    rels = jnp.pad(relation_type.astype(jnp.int32), (0, pad))
    score = _sc_score(xnp, trig, heads, tails, rels)
    return score[:N_EDGES]


# speedup vs baseline: 1.0306x; 1.0306x over previous
import functools

import jax
import jax.numpy as jnp
import numpy as np
from jax import lax
from jax.experimental import pallas as pl
from jax.experimental.pallas import tpu as pltpu
from jax.experimental.pallas import tpu_sc as plsc

EMBEDDING_DIM = 256
HALF = EMBEDDING_DIM // 2
NUM_RELATIONS = 64
N_NODES = 10000
N_EDGES = 160000
EPS = 1e-08

NUM_CORES = 2
NUM_SUBCORES = 16
NUM_WORKERS = NUM_CORES * NUM_SUBCORES
LANES = 16

PAD_EDGES = 163840
EDGES_PER_WORKER = PAD_EDGES // NUM_WORKERS
CHUNK = 80
NUM_CHUNKS = EDGES_PER_WORKER // CHUNK
PAIRS = NUM_CHUNKS // 2
GROUPS = CHUNK // LANES
DIM_GROUPS = HALF // LANES

_RSQRT_MAGIC = np.int32(0x5F3759DF)
_HI_MASK = np.int32(-65536)


def _tc_prep(x_ref, w_ref, xnp_ref, trig_ref):
    x = x_ref[...]
    norm = jnp.sqrt(jnp.sum(x * x, axis=1, keepdims=True))
    xn = x / jnp.maximum(norm, 1e-12)
    re = xn[:, :HALF]
    im = xn[:, HALF:]
    rb = lax.bitcast_convert_type(re, jnp.uint32)
    ib = lax.bitcast_convert_type(im, jnp.uint32)
    rb = rb + 0x7FFF + ((rb >> 16) & 1)
    ib = ib + 0x7FFF + ((ib >> 16) & 1)
    packed = (ib & np.uint32(0xFFFF0000)) | (rb >> 16)
    xnp_ref[...] = lax.bitcast_convert_type(packed, jnp.int32)
    w = w_ref[...]
    trig_ref[...] = jnp.concatenate([jnp.cos(w), jnp.sin(w)], axis=1)


def _approx_sqrt(sq):
    i = lax.bitcast_convert_type(sq, jnp.int32)
    i = _RSQRT_MAGIC - lax.shift_right_logical(i, 1)
    r = lax.bitcast_convert_type(i, jnp.float32)
    r = r * (1.5 - (sq * 0.5) * r * r)
    return sq * r


def _unpack(word):
    lo = lax.bitcast_convert_type(lax.shift_left(word, 16), jnp.float32)
    hi = lax.bitcast_convert_type(word & _HI_MASK, jnp.float32)
    return lo, hi


NBUF = 4


def _sc_body(xnp_hbm, trig_hbm, heads_hbm, tails_hbm, rels_hbm, out_hbm,
             hidx, tidx, ridx, trig_v,
             hrows0, trows0, hrows1, trows1,
             hrows2, trows2, hrows3, trows3, oacc,
             sem_h0, sem_t0, sem_h1, sem_t1,
             sem_h2, sem_t2, sem_h3, sem_t3):
    hbufs = (hrows0, hrows1, hrows2, hrows3)
    tbufs = (trows0, trows1, trows2, trows3)
    hsems = (sem_h0, sem_h1, sem_h2, sem_h3)
    tsems = (sem_t0, sem_t1, sem_t2, sem_t3)
    wid = lax.axis_index("s") * NUM_CORES + lax.axis_index("c")
    base = wid * EDGES_PER_WORKER
    pltpu.sync_copy(heads_hbm.at[pl.ds(base, EDGES_PER_WORKER)], hidx)
    pltpu.sync_copy(tails_hbm.at[pl.ds(base, EDGES_PER_WORKER)], tidx)
    pltpu.sync_copy(rels_hbm.at[pl.ds(base, EDGES_PER_WORKER)], ridx)
    pltpu.sync_copy(trig_hbm, trig_v)
    lane = lax.broadcasted_iota(jnp.int32, (LANES,), 0)

    def issue(ci, hrows, trows, sem_h, sem_t):
        off = ci * CHUNK
        pltpu.async_copy(xnp_hbm.at[hidx.at[pl.ds(off, CHUNK)]], hrows, sem_h)
        pltpu.async_copy(xnp_hbm.at[tidx.at[pl.ds(off, CHUNK)]], trows, sem_t)

    def wait(hrows, trows, sem_h, sem_t):
        pltpu.make_async_copy(xnp_hbm.at[hidx.at[pl.ds(0, CHUNK)]], hrows,
                              sem_h).wait()
        pltpu.make_async_copy(xnp_hbm.at[tidx.at[pl.ds(0, CHUNK)]], trows,
                              sem_t).wait()

    def compute(ci, hrows, trows):
        off = ci * CHUNK

        def group_body(gi, gcarry):
            goff = gi * LANES
            relvec = ridx[pl.ds(off + goff, LANES)]
            vec = jnp.zeros((LANES,), jnp.float32)
            for e in range(LANES):
                eidx = goff + e
                rel = relvec[e]
                acc = jnp.zeros((LANES,), jnp.float32)
                for d in range(DIM_GROUPS):
                    sl = pl.ds(d * LANES, LANES)
                    sl_im = pl.ds(HALF + d * LANES, LANES)
                    hre, him = _unpack(hrows[eidx, sl])
                    tre, tim = _unpack(trows[eidx, sl])
                    cw = trig_v[rel, sl]
                    sw = trig_v[rel, sl_im]
                    re_d = hre * cw - him * sw - tre
                    im_d = hre * sw + him * cw - tim
                    sq = re_d * re_d + im_d * im_d + EPS
                    acc = acc + _approx_sqrt(sq)
                s = jnp.sum(acc * (-1.0 / HALF))
                vec = jnp.where(lane == e, s, vec)
            oacc[pl.ds(off + goff, LANES)] = vec
            return gcarry

        lax.fori_loop(0, GROUPS, group_body, 0)

    for b in range(NBUF):
        issue(b, hbufs[b], tbufs[b], hsems[b], tsems[b])

    def quad_body(qi, carry):
        ci = qi * NBUF
        for b in range(NBUF):
            wait(hbufs[b], tbufs[b], hsems[b], tsems[b])
            compute(ci + b, hbufs[b], tbufs[b])

            @pl.when(qi < NUM_CHUNKS // NBUF - 1)
            def _():
                issue(ci + b + NBUF, hbufs[b], tbufs[b], hsems[b], tsems[b])

        return carry

    lax.fori_loop(0, NUM_CHUNKS // NBUF, quad_body, 0)
    pltpu.sync_copy(oacc, out_hbm.at[pl.ds(base, EDGES_PER_WORKER)])


_sc_score = functools.partial(
    pl.kernel,
    out_type=jax.ShapeDtypeStruct((PAD_EDGES,), jnp.float32),
    mesh=plsc.VectorSubcoreMesh(core_axis_name="c", subcore_axis_name="s"),
    compiler_params=pltpu.CompilerParams(needs_layout_passes=False),
    scratch_types=[
        pltpu.VMEM((EDGES_PER_WORKER,), jnp.int32),
        pltpu.VMEM((EDGES_PER_WORKER,), jnp.int32),
        pltpu.VMEM((EDGES_PER_WORKER,), jnp.int32),
        pltpu.VMEM((NUM_RELATIONS, EMBEDDING_DIM), jnp.float32),
        pltpu.VMEM((CHUNK, HALF), jnp.int32),
        pltpu.VMEM((CHUNK, HALF), jnp.int32),
        pltpu.VMEM((CHUNK, HALF), jnp.int32),
        pltpu.VMEM((CHUNK, HALF), jnp.int32),
        pltpu.VMEM((CHUNK, HALF), jnp.int32),
        pltpu.VMEM((CHUNK, HALF), jnp.int32),
        pltpu.VMEM((CHUNK, HALF), jnp.int32),
        pltpu.VMEM((CHUNK, HALF), jnp.int32),
        pltpu.VMEM((EDGES_PER_WORKER,), jnp.float32),
        pltpu.SemaphoreType.DMA,
        pltpu.SemaphoreType.DMA,
        pltpu.SemaphoreType.DMA,
        pltpu.SemaphoreType.DMA,
        pltpu.SemaphoreType.DMA,
        pltpu.SemaphoreType.DMA,
        pltpu.SemaphoreType.DMA,
        pltpu.SemaphoreType.DMA,
    ],
)(_sc_body)


def kernel(node_embeddings, edge_index, relation_type, rel_weight):
    xnp, trig = pl.pallas_call(
        _tc_prep,
        out_shape=[
            jax.ShapeDtypeStruct((N_NODES, HALF), jnp.int32),
            jax.ShapeDtypeStruct((NUM_RELATIONS, EMBEDDING_DIM), jnp.float32),
        ],
    )(node_embeddings, rel_weight)
    pad = PAD_EDGES - N_EDGES
    heads = jnp.pad(edge_index[0].astype(jnp.int32), (0, pad))
    tails = jnp.pad(edge_index[1].astype(jnp.int32), (0, pad))
    rels = jnp.pad(relation_type.astype(jnp.int32), (0, pad))
    score = _sc_score(xnp, trig, heads, tails, rels)
    return score[:N_EDGES]
